# Initial kernel scaffold; baseline (speedup 1.0000x reference)
import jax
import jax.numpy as jnp
from jax.experimental import pallas as pl


def kernel(tokens, labels):
    raise NotImplementedError("write your pallas kernel here")


# speedup vs baseline: 1.2736x; 1.2736x over previous
import jax
import jax.numpy as jnp
from jax.experimental import pallas as pl

_TEMPERATURE = 0.07
_NEG_INF = -1e30


def _loss_kernel(tok_ref, lc_ref, lr_ref, out_ref):
    lc = lc_ref[:, :]
    lr = lr_ref[:, :]
    B = lc.shape[0]
    same = lc == lr
    row = jax.lax.broadcasted_iota(jnp.int32, (B, B), 0)
    col = jax.lax.broadcasted_iota(jnp.int32, (B, B), 1)
    eye = row == col
    posf = jnp.where(jnp.logical_and(same, jnp.logical_not(eye)),
                     jnp.float32(1.0), jnp.float32(0.0))
    pos_count = jnp.sum(posf, axis=1, keepdims=True)
    valid = pos_count > 0.0
    inv_cnt = 1.0 / jnp.maximum(pos_count, 1.0)

    total = jnp.float32(0.0)
    inv_t = jnp.float32(1.0 / _TEMPERATURE)
    for j in range(tok_ref.shape[0]):
        tj = tok_ref[j]
        S = jax.lax.dot_general(
            tj, tj, (((1,), (1,)), ((), ())),
            preferred_element_type=jnp.float32) * inv_t
        Sm = jnp.where(eye, jnp.float32(_NEG_INF), S)
        m = jnp.max(Sm, axis=1, keepdims=True)
        lse = m + jnp.log(jnp.sum(jnp.exp(Sm - m), axis=1, keepdims=True))
        pos_sum = jnp.sum(posf * S, axis=1, keepdims=True) - pos_count * lse
        per_anchor = pos_sum * inv_cnt
        total = total + jnp.sum(jnp.where(valid, per_anchor, 0.0))

    nvalid = jnp.sum(jnp.where(valid, jnp.float32(1.0), jnp.float32(0.0)))
    m_f = jnp.float32(tok_ref.shape[0])
    out_ref[:, :] = (total / (-m_f * nvalid)).reshape(1, 1)


def kernel(tokens, labels):
    if tokens.ndim == 2:
        tokens = tokens[:, None, :]
    tokens = jnp.transpose(tokens, (1, 0, 2)).astype(jnp.float32)
    labels = labels.astype(jnp.int32)
    B = tokens.shape[1]
    lc = labels.reshape(B, 1)
    lr = labels.reshape(1, B)
    out = pl.pallas_call(
        _loss_kernel,
        out_shape=jax.ShapeDtypeStruct((1, 1), jnp.float32),
    )(tokens, lc, lr)
    return out[0, 0]
